# Initial kernel scaffold; baseline (speedup 1.0000x reference)
import jax
import jax.numpy as jnp
from jax.experimental import pallas as pl


def kernel(x, edge_index, W1, b1, W2, b2):
    raise NotImplementedError("write your pallas kernel here")


# speedup vs baseline: 3.9052x; 3.9052x over previous
import jax
import jax.numpy as jnp
from jax import lax
from jax.experimental import pallas as pl
from jax.experimental.pallas import tpu as pltpu
from jax.experimental.pallas import tpu_sc as plsc

N_NODES = 10000
D = 128
NC, NS = 2, 16
NW = NC * NS
C = 128
NCHUNK = 80
ROUNDS = 5
CH = NCHUNK // ROUNDS
E_PAD = NW * NCHUNK * C
N_PAD = 10240
TRASH = N_NODES
ROWS_PER_SUB = N_PAD // NS
RCHUNKS = ROWS_PER_SUB // C
HROWS = N_PAD // C
FBASE = NC * N_PAD


def _sc_body(xt, pki, outf, src_v, dst_v, buf0, buf1, hist, acc, sem0, sem1):
  c = lax.axis_index("c")
  s = lax.axis_index("s")
  wid = c * NS + s
  base = s * ROWS_PER_SUB
  z16 = jnp.zeros((16,), jnp.float32)

  @pl.loop(0, C)
  def _(r):
    for k in range(D // 16):
      buf0[r, pl.ds(k * 16, 16)] = z16

  @pl.loop(0, HROWS)
  def _(r):
    for k in range(C // 16):
      hist[r, pl.ds(k * 16, 16)] = z16

  for t in range(RCHUNKS):
    pltpu.sync_copy(buf0, acc.at[pl.ds(base + t * C, C)])

  plsc.subcore_barrier()

  for r in range(ROUNDS):
    pltpu.sync_copy(pki.at[pl.ds(wid * NCHUNK + r * CH, CH)], dst_v)

    @pl.loop(0, CH)
    def _(j):
      for k in range(C // 16):
        p = dst_v[j, pl.ds(k * 16, 16)]
        v = lax.shift_right_logical(p, 14)
        src_v[j, pl.ds(k * 16, 16)] = jnp.bitwise_and(p, 16383)
        dst_v[j, pl.ds(k * 16, 16)] = v
        cnt, lastm = plsc.scan_count(v)
        plsc.addupdate_scatter(
            hist, [lax.shift_right_logical(v, 7), jnp.bitwise_and(v, 127)],
            cnt.astype(jnp.float32), mask=lastm)

    pltpu.async_copy(xt.at[src_v.at[0]], buf0, sem0)
    pltpu.async_copy(xt.at[src_v.at[1]], buf1, sem1)

    @pl.loop(0, CH // 2)
    def _(i):
      j0 = i * 2
      for b, (buf, sem) in enumerate(((buf0, sem0), (buf1, sem1))):
        j = j0 + b
        pltpu.make_async_copy(xt.at[src_v.at[j]], buf, sem).wait()
        pltpu.sync_copy(buf, acc.at[dst_v.at[j]], add=True)
        jn = jnp.minimum(j + 2, CH - 1)
        pltpu.async_copy(xt.at[src_v.at[jn]], buf, sem)

    pltpu.make_async_copy(xt.at[src_v.at[CH - 1]], buf0, sem0).wait()
    pltpu.make_async_copy(xt.at[src_v.at[CH - 1]], buf1, sem1).wait()

  pltpu.sync_copy(hist, outf.at[pl.ds(FBASE + wid * HROWS, HROWS)])

  plsc.subcore_barrier()

  for t in range(RCHUNKS):
    pltpu.sync_copy(acc.at[pl.ds(base + t * C, C)], buf0)
    pltpu.sync_copy(buf0, outf.at[pl.ds(c * N_PAD + base + t * C, C)])


_sc_scatter_cache = []


def _sc_scatter(*args):
  if not _sc_scatter_cache:
    mesh = plsc.VectorSubcoreMesh(
        core_axis_name="c", subcore_axis_name="s",
        num_cores=NC, num_subcores=NS)
    _sc_scatter_cache.append(pl.kernel(
        _sc_body,
        out_type=jax.ShapeDtypeStruct((FBASE + NW * HROWS, D), jnp.float32),
        mesh=mesh,
        compiler_params=pltpu.CompilerParams(needs_layout_passes=False),
        scratch_types=[
            pltpu.VMEM((CH, C), jnp.int32),
            pltpu.VMEM((CH, C), jnp.int32),
            pltpu.VMEM((C, D), jnp.float32),
            pltpu.VMEM((C, D), jnp.float32),
            pltpu.VMEM((HROWS, C), jnp.float32),
            pltpu.VMEM_SHARED((N_PAD, D), jnp.float32),
            pltpu.SemaphoreType.DMA,
            pltpu.SemaphoreType.DMA,
        ],
    ))
  return _sc_scatter_cache[0](*args)


BN = 1024


def _mlp_body(p_ref, dp_ref, w1t_ref, b1_ref, w2t_ref, b2_ref, o_ref):
  agr = p_ref[0] + p_ref[1]
  deg = jnp.sum(dp_ref[...], axis=0)[:, None]
  xn = agr / (deg + 1e-8)
  h = jnp.tanh(
      jnp.dot(xn, w1t_ref[...], preferred_element_type=jnp.float32)
      + b1_ref[...])
  o_ref[...] = (
      jnp.dot(h, w2t_ref[...], preferred_element_type=jnp.float32)
      + b2_ref[...])


def _mlp(p, dp, w1t, b1, w2t, b2):
  grid = N_PAD // BN
  return pl.pallas_call(
      _mlp_body,
      grid=(grid,),
      in_specs=[
          pl.BlockSpec((NC, BN, D), lambda i: (0, i, 0)),
          pl.BlockSpec((NW, BN), lambda i: (0, i)),
          pl.BlockSpec((D, D), lambda i: (0, 0)),
          pl.BlockSpec((1, D), lambda i: (0, 0)),
          pl.BlockSpec((D, D), lambda i: (0, 0)),
          pl.BlockSpec((1, D), lambda i: (0, 0)),
      ],
      out_specs=pl.BlockSpec((BN, D), lambda i: (i, 0)),
      out_shape=jax.ShapeDtypeStruct((N_PAD, D), jnp.float32),
  )(p, dp, w1t, b1, w2t, b2)


def kernel(x, edge_index, W1, b1, W2, b2):
  src = edge_index[0].astype(jnp.int32)
  dst = edge_index[1].astype(jnp.int32)
  e = src.shape[0]
  pad = E_PAD - e
  packed = jnp.bitwise_or(src, lax.shift_left(dst, 14))
  packed = jnp.concatenate(
      [packed, jnp.full((pad,), TRASH << 14, jnp.int32)])
  packed = packed.reshape(NW * NCHUNK, C)
  outf = _sc_scatter(x, packed)
  p = outf[:FBASE].reshape(NC, N_PAD, D)
  dp = outf[FBASE:].reshape(NW, N_PAD)
  out = _mlp(p, dp, W1.T, b1.reshape(1, D), W2.T, b2.reshape(1, D))
  return out[:N_NODES]


# speedup vs baseline: 12.0524x; 3.0862x over previous
import jax
import jax.numpy as jnp
from jax import lax
from jax.experimental import pallas as pl
from jax.experimental.pallas import tpu as pltpu
from jax.experimental.pallas import tpu_sc as plsc

N_NODES = 10000
D = 128
NC, NS = 2, 16
NW = NC * NS
C = 128
NCHUNK = 80
ROUNDS = 5
CH = NCHUNK // ROUNDS
E_PAD = NW * NCHUNK * C
N_PAD = 10240
TRASH = N_NODES
ROWS_PER_SUB = N_PAD // NS
RCHUNKS = ROWS_PER_SUB // C
HROWS = N_PAD // C
FBASE = NC * N_PAD


def _sc_body(xt, pki, outf, src_v, dst_v, buf0, buf1, hist, acc, sem0, sem1):
  c = lax.axis_index("c")
  s = lax.axis_index("s")
  wid = c * NS + s
  base = s * ROWS_PER_SUB
  z16 = jnp.zeros((16,), jnp.float32)

  @pl.loop(0, C)
  def _(r):
    for k in range(D // 16):
      buf0[r, pl.ds(k * 16, 16)] = z16

  @pl.loop(0, HROWS)
  def _(r):
    for k in range(C // 16):
      hist[r, pl.ds(k * 16, 16)] = z16

  for t in range(RCHUNKS):
    pltpu.sync_copy(buf0, acc.at[pl.ds(base + t * C, C)])

  plsc.subcore_barrier()

  for r in range(ROUNDS):
    pltpu.sync_copy(pki.at[pl.ds(wid * NCHUNK + r * CH, CH)], dst_v)

    @pl.loop(0, CH)
    def _(j):
      for k in range(C // 16):
        p = dst_v[j, pl.ds(k * 16, 16)]
        v = lax.shift_right_logical(p, 14)
        src_v[j, pl.ds(k * 16, 16)] = jnp.bitwise_and(p, 16383)
        dst_v[j, pl.ds(k * 16, 16)] = v
        cnt, lastm = plsc.scan_count(v)
        plsc.addupdate_scatter(
            hist, [lax.shift_right_logical(v, 7), jnp.bitwise_and(v, 127)],
            cnt.astype(jnp.float32), mask=lastm)

    pltpu.async_copy(xt.at[src_v.at[0]], buf0, sem0)
    pltpu.async_copy(xt.at[src_v.at[1]], buf1, sem1)

    @pl.loop(0, CH // 2)
    def _(i):
      j0 = i * 2
      for b, (buf, sem) in enumerate(((buf0, sem0), (buf1, sem1))):
        j = j0 + b
        pltpu.make_async_copy(xt.at[src_v.at[j]], buf, sem).wait()
        pltpu.sync_copy(buf, acc.at[dst_v.at[j]], add=True)
        jn = jnp.minimum(j + 2, CH - 1)
        pltpu.async_copy(xt.at[src_v.at[jn]], buf, sem)

    pltpu.make_async_copy(xt.at[src_v.at[CH - 1]], buf0, sem0).wait()
    pltpu.make_async_copy(xt.at[src_v.at[CH - 1]], buf1, sem1).wait()

  pltpu.sync_copy(hist, outf.at[pl.ds(FBASE + wid * HROWS, HROWS)])

  plsc.subcore_barrier()

  for t in range(RCHUNKS):
    pltpu.sync_copy(acc.at[pl.ds(base + t * C, C)], buf0)
    pltpu.sync_copy(buf0, outf.at[pl.ds(c * N_PAD + base + t * C, C)])


_sc_scatter_cache = []


def _sc_scatter(*args):
  if not _sc_scatter_cache:
    mesh = plsc.VectorSubcoreMesh(
        core_axis_name="c", subcore_axis_name="s",
        num_cores=NC, num_subcores=NS)
    _sc_scatter_cache.append(pl.kernel(
        _sc_body,
        out_type=jax.ShapeDtypeStruct((FBASE + NW * HROWS, D), jnp.float32),
        mesh=mesh,
        compiler_params=pltpu.CompilerParams(needs_layout_passes=False),
        scratch_types=[
            pltpu.VMEM((CH, C), jnp.int32),
            pltpu.VMEM((CH, C), jnp.int32),
            pltpu.VMEM((C, D), jnp.float32),
            pltpu.VMEM((C, D), jnp.float32),
            pltpu.VMEM((HROWS, C), jnp.float32),
            pltpu.VMEM_SHARED((N_PAD, D), jnp.float32),
            pltpu.SemaphoreType.DMA,
            pltpu.SemaphoreType.DMA,
        ],
    ))
  return _sc_scatter_cache[0](*args)


BN = 1024


def _mlp_body(p_ref, dp_ref, w1t_ref, b1_ref, w2t_ref, b2_ref, o_ref):
  agr = p_ref[0] + p_ref[1]
  deg = jnp.sum(dp_ref[...], axis=0)[:, None]
  xn = agr / (deg + 1e-8)
  h = jnp.tanh(
      jnp.dot(xn, w1t_ref[...], preferred_element_type=jnp.float32)
      + b1_ref[...])
  o_ref[...] = (
      jnp.dot(h, w2t_ref[...], preferred_element_type=jnp.float32)
      + b2_ref[...])


def _mlp(p, dp, w1t, b1, w2t, b2):
  grid = N_PAD // BN
  return pl.pallas_call(
      _mlp_body,
      grid=(grid,),
      in_specs=[
          pl.BlockSpec((NC, BN, D), lambda i: (0, i, 0)),
          pl.BlockSpec((NW, BN), lambda i: (0, i)),
          pl.BlockSpec((D, D), lambda i: (0, 0)),
          pl.BlockSpec((1, D), lambda i: (0, 0)),
          pl.BlockSpec((D, D), lambda i: (0, 0)),
          pl.BlockSpec((1, D), lambda i: (0, 0)),
      ],
      out_specs=pl.BlockSpec((BN, D), lambda i: (i, 0)),
      out_shape=jax.ShapeDtypeStruct((N_PAD, D), jnp.float32),
  )(p, dp, w1t, b1, w2t, b2)


def kernel(x, edge_index, W1, b1, W2, b2):
  src = edge_index[0].astype(jnp.int32)
  dst = edge_index[1].astype(jnp.int32)
  e = src.shape[0]
  pad = E_PAD - e
  packed = jnp.bitwise_or(src, lax.shift_left(dst, 14))
  pad_i = jnp.arange(pad, dtype=jnp.int32)
  pad_dst = TRASH + pad_i % (N_PAD - N_NODES)
  pad_src = pad_i % N_NODES
  packed = jnp.concatenate(
      [packed, jnp.bitwise_or(pad_src, lax.shift_left(pad_dst, 14))])
  packed = packed.reshape(NW * NCHUNK, C)
  outf = _sc_scatter(x, packed)
  p = outf[:FBASE].reshape(NC, N_PAD, D)
  dp = outf[FBASE:].reshape(NW, N_PAD)
  out = _mlp(p, dp, W1.T, b1.reshape(1, D), W2.T, b2.reshape(1, D))
  return out[:N_NODES]


# speedup vs baseline: 12.5552x; 1.0417x over previous
import jax
import jax.numpy as jnp
from jax import lax
from jax.experimental import pallas as pl
from jax.experimental.pallas import tpu as pltpu
from jax.experimental.pallas import tpu_sc as plsc

N_NODES = 10000
D = 128
NC, NS = 2, 16
NW = NC * NS
C = 128
NCHUNK = 80
ROUNDS = 5
CH = NCHUNK // ROUNDS
E_PAD = NW * NCHUNK * C
N_PAD = 10240
TRASH = N_NODES
ROWS_PER_SUB = N_PAD // NS
RCHUNKS = ROWS_PER_SUB // C
HROWS = N_PAD // C
FBASE = NC * N_PAD


def _sc_body(xt, srci, dsti, outf, src_v, dst_v, buf0, buf1, hist, acc,
             sem0, sem1, ssem0, ssem1):
  c = lax.axis_index("c")
  s = lax.axis_index("s")
  wid = c * NS + s
  base = s * ROWS_PER_SUB
  z16 = jnp.zeros((16,), jnp.float32)

  @pl.loop(0, C)
  def _(r):
    for k in range(D // 16):
      buf0[r, pl.ds(k * 16, 16)] = z16

  @pl.loop(0, HROWS)
  def _(r):
    for k in range(C // 16):
      hist[r, pl.ds(k * 16, 16)] = z16

  for t in range(RCHUNKS):
    pltpu.sync_copy(buf0, acc.at[pl.ds(base + t * C, C)])

  plsc.subcore_barrier()

  for r in range(ROUNDS):
    pltpu.sync_copy(srci.at[pl.ds(wid * NCHUNK + r * CH, CH)], src_v)
    pltpu.sync_copy(dsti.at[pl.ds(wid * NCHUNK + r * CH, CH)], dst_v)

    pltpu.async_copy(xt.at[src_v.at[0]], buf0, sem0)
    pltpu.async_copy(xt.at[src_v.at[1]], buf1, sem1)

    @pl.loop(0, CH // 2)
    def _(i):
      j0 = i * 2
      for b, (buf, sem, ssem) in enumerate(
          ((buf0, sem0, ssem0), (buf1, sem1, ssem1))):
        j = j0 + b
        pltpu.make_async_copy(xt.at[src_v.at[j]], buf, sem).wait()
        pltpu.async_copy(buf, acc.at[dst_v.at[j]], ssem, add=True)
        for k in range(C // 16):
          v = dst_v[j, pl.ds(k * 16, 16)]
          cnt, lastm = plsc.scan_count(v)
          plsc.addupdate_scatter(
              hist,
              [lax.shift_right_logical(v, 7), jnp.bitwise_and(v, 127)],
              cnt.astype(jnp.float32), mask=lastm)
        pltpu.make_async_copy(buf, acc.at[dst_v.at[j]], ssem).wait()
        jn = jnp.minimum(j + 2, CH - 1)
        pltpu.async_copy(xt.at[src_v.at[jn]], buf, sem)

    pltpu.make_async_copy(xt.at[src_v.at[CH - 1]], buf0, sem0).wait()
    pltpu.make_async_copy(xt.at[src_v.at[CH - 1]], buf1, sem1).wait()

  pltpu.sync_copy(hist, outf.at[pl.ds(FBASE + wid * HROWS, HROWS)])

  plsc.subcore_barrier()

  for t in range(RCHUNKS):
    pltpu.sync_copy(acc.at[pl.ds(base + t * C, C)], buf0)
    pltpu.sync_copy(buf0, outf.at[pl.ds(c * N_PAD + base + t * C, C)])


_sc_scatter_cache = []


def _sc_scatter(*args):
  if not _sc_scatter_cache:
    mesh = plsc.VectorSubcoreMesh(
        core_axis_name="c", subcore_axis_name="s",
        num_cores=NC, num_subcores=NS)
    _sc_scatter_cache.append(pl.kernel(
        _sc_body,
        out_type=jax.ShapeDtypeStruct((FBASE + NW * HROWS, D), jnp.float32),
        mesh=mesh,
        compiler_params=pltpu.CompilerParams(needs_layout_passes=False),
        scratch_types=[
            pltpu.VMEM((CH, C), jnp.int32),
            pltpu.VMEM((CH, C), jnp.int32),
            pltpu.VMEM((C, D), jnp.float32),
            pltpu.VMEM((C, D), jnp.float32),
            pltpu.VMEM((HROWS, C), jnp.float32),
            pltpu.VMEM_SHARED((N_PAD, D), jnp.float32),
            pltpu.SemaphoreType.DMA,
            pltpu.SemaphoreType.DMA,
            pltpu.SemaphoreType.DMA,
            pltpu.SemaphoreType.DMA,
        ],
    ))
  return _sc_scatter_cache[0](*args)


BN = 1024


def _mlp_body(p_ref, dp_ref, w1t_ref, b1_ref, w2t_ref, b2_ref, o_ref):
  agr = p_ref[0] + p_ref[1]
  deg = jnp.sum(dp_ref[...], axis=0)[:, None]
  xn = agr / (deg + 1e-8)
  h = jnp.tanh(
      jnp.dot(xn, w1t_ref[...], preferred_element_type=jnp.float32)
      + b1_ref[...])
  o_ref[...] = (
      jnp.dot(h, w2t_ref[...], preferred_element_type=jnp.float32)
      + b2_ref[...])


def _mlp(p, dp, w1t, b1, w2t, b2):
  grid = N_PAD // BN
  return pl.pallas_call(
      _mlp_body,
      grid=(grid,),
      in_specs=[
          pl.BlockSpec((NC, BN, D), lambda i: (0, i, 0)),
          pl.BlockSpec((NW, BN), lambda i: (0, i)),
          pl.BlockSpec((D, D), lambda i: (0, 0)),
          pl.BlockSpec((1, D), lambda i: (0, 0)),
          pl.BlockSpec((D, D), lambda i: (0, 0)),
          pl.BlockSpec((1, D), lambda i: (0, 0)),
      ],
      out_specs=pl.BlockSpec((BN, D), lambda i: (i, 0)),
      out_shape=jax.ShapeDtypeStruct((N_PAD, D), jnp.float32),
  )(p, dp, w1t, b1, w2t, b2)


def kernel(x, edge_index, W1, b1, W2, b2):
  src = edge_index[0].astype(jnp.int32)
  dst = edge_index[1].astype(jnp.int32)
  e = src.shape[0]
  pad = E_PAD - e
  pad_i = jnp.arange(pad, dtype=jnp.int32)
  pad_dst = TRASH + pad_i % (N_PAD - N_NODES)
  pad_src = pad_i % N_NODES
  src_p = jnp.concatenate([src, pad_src]).reshape(NW * NCHUNK, C)
  dst_p = jnp.concatenate([dst, pad_dst]).reshape(NW * NCHUNK, C)
  outf = _sc_scatter(x, src_p, dst_p)
  p = outf[:FBASE].reshape(NC, N_PAD, D)
  dp = outf[FBASE:].reshape(NW, N_PAD)
  out = _mlp(p, dp, W1.T, b1.reshape(1, D), W2.T, b2.reshape(1, D))
  return out[:N_NODES]


# speedup vs baseline: 13.1561x; 1.0479x over previous
import jax
import jax.numpy as jnp
from jax import lax
from jax.experimental import pallas as pl
from jax.experimental.pallas import tpu as pltpu
from jax.experimental.pallas import tpu_sc as plsc

N_NODES = 10000
D = 128
NC, NS = 2, 16
NW = NC * NS
C = 128
NCHUNK = 80
ROUNDS = 5
CH = NCHUNK // ROUNDS
E_PAD = NW * NCHUNK * C
N_PAD = 10240
TRASH = N_NODES
ROWS_PER_SUB = N_PAD // NS
RCHUNKS = ROWS_PER_SUB // C
HROWS = N_PAD // C
FBASE = NC * N_PAD


def _sc_body(xt, srci, dsti, outf, src_v, dst_v, buf0, buf1, hist, acc,
             sem0, sem1, ssem0, ssem1):
  c = lax.axis_index("c")
  s = lax.axis_index("s")
  wid = c * NS + s
  base = s * ROWS_PER_SUB
  z16 = jnp.zeros((16,), jnp.float32)

  @pl.loop(0, C)
  def _(r):
    for k in range(D // 16):
      buf0[r, pl.ds(k * 16, 16)] = z16

  @pl.loop(0, HROWS)
  def _(r):
    for k in range(C // 16):
      hist[r, pl.ds(k * 16, 16)] = z16

  for t in range(RCHUNKS):
    pltpu.sync_copy(buf0, acc.at[pl.ds(base + t * C, C)])

  plsc.subcore_barrier()

  for r in range(ROUNDS):
    pltpu.sync_copy(srci.at[pl.ds(wid * NCHUNK + r * CH, CH)], src_v)
    pltpu.sync_copy(dsti.at[pl.ds(wid * NCHUNK + r * CH, CH)], dst_v)

    pltpu.async_copy(xt.at[src_v.at[0]], buf0, sem0)
    pltpu.async_copy(xt.at[src_v.at[1]], buf1, sem1)

    @pl.loop(0, CH // 2)
    def _(i):
      j0 = i * 2
      for b, (buf, sem, ssem) in enumerate(
          ((buf0, sem0, ssem0), (buf1, sem1, ssem1))):
        j = j0 + b
        pltpu.make_async_copy(xt.at[src_v.at[j]], buf, sem).wait()
        pltpu.async_copy(buf, acc.at[dst_v.at[j]], ssem, add=True)
        for k in range(C // 16):
          v = dst_v[j, pl.ds(k * 16, 16)]
          cnt, lastm = plsc.scan_count(v)
          plsc.addupdate_scatter(
              hist,
              [lax.shift_right_logical(v, 7), jnp.bitwise_and(v, 127)],
              cnt.astype(jnp.float32), mask=lastm)
        pltpu.make_async_copy(buf, acc.at[dst_v.at[j]], ssem).wait()
        jn = jnp.minimum(j + 2, CH - 1)
        pltpu.async_copy(xt.at[src_v.at[jn]], buf, sem)

    pltpu.make_async_copy(xt.at[src_v.at[CH - 1]], buf0, sem0).wait()
    pltpu.make_async_copy(xt.at[src_v.at[CH - 1]], buf1, sem1).wait()

  pltpu.sync_copy(hist, outf.at[pl.ds(FBASE + wid * HROWS, HROWS)])

  plsc.subcore_barrier()

  for t in range(RCHUNKS):
    pltpu.sync_copy(acc.at[pl.ds(base + t * C, C)], buf0)
    pltpu.sync_copy(buf0, outf.at[pl.ds(c * N_PAD + base + t * C, C)])


_sc_scatter_cache = []


def _sc_scatter(*args):
  if not _sc_scatter_cache:
    mesh = plsc.VectorSubcoreMesh(
        core_axis_name="c", subcore_axis_name="s",
        num_cores=NC, num_subcores=NS)
    _sc_scatter_cache.append(pl.kernel(
        _sc_body,
        out_type=jax.ShapeDtypeStruct((FBASE + NW * HROWS, D), jnp.float32),
        mesh=mesh,
        compiler_params=pltpu.CompilerParams(needs_layout_passes=False),
        scratch_types=[
            pltpu.VMEM((CH, C), jnp.int32),
            pltpu.VMEM((CH, C), jnp.int32),
            pltpu.VMEM((C, D), jnp.float32),
            pltpu.VMEM((C, D), jnp.float32),
            pltpu.VMEM((HROWS, C), jnp.float32),
            pltpu.VMEM_SHARED((N_PAD, D), jnp.float32),
            pltpu.SemaphoreType.DMA,
            pltpu.SemaphoreType.DMA,
            pltpu.SemaphoreType.DMA,
            pltpu.SemaphoreType.DMA,
        ],
    ))
  return _sc_scatter_cache[0](*args)


BN = 1024


def _mlp_body(p0_ref, p1_ref, dp_ref, w1t_ref, b1_ref, w2t_ref, b2_ref,
              o_ref):
  agr = p0_ref[...] + p1_ref[...]
  deg = jnp.sum(dp_ref[...], axis=0)[:, None]
  xn = agr / (deg + 1e-8)
  h = jnp.tanh(
      jnp.dot(xn, w1t_ref[...], preferred_element_type=jnp.float32)
      + b1_ref[...])
  o_ref[...] = (
      jnp.dot(h, w2t_ref[...], preferred_element_type=jnp.float32)
      + b2_ref[...])


def _mlp(outf, dp, w1t, b1, w2t, b2):
  grid = N_PAD // BN
  return pl.pallas_call(
      _mlp_body,
      grid=(grid,),
      in_specs=[
          pl.BlockSpec((BN, D), lambda i: (i, 0)),
          pl.BlockSpec((BN, D), lambda i: (N_PAD // BN + i, 0)),
          pl.BlockSpec((NW, BN), lambda i: (0, i)),
          pl.BlockSpec((D, D), lambda i: (0, 0)),
          pl.BlockSpec((1, D), lambda i: (0, 0)),
          pl.BlockSpec((D, D), lambda i: (0, 0)),
          pl.BlockSpec((1, D), lambda i: (0, 0)),
      ],
      out_specs=pl.BlockSpec((BN, D), lambda i: (i, 0)),
      out_shape=jax.ShapeDtypeStruct((N_PAD, D), jnp.float32),
  )(outf, outf, dp, w1t, b1, w2t, b2)


def kernel(x, edge_index, W1, b1, W2, b2):
  src = edge_index[0].astype(jnp.int32)
  dst = edge_index[1].astype(jnp.int32)
  e = src.shape[0]
  pad = E_PAD - e
  pad_i = jnp.arange(pad, dtype=jnp.int32)
  pad_dst = TRASH + pad_i % (N_PAD - N_NODES)
  pad_src = pad_i % N_NODES
  src_p = jnp.concatenate([src, pad_src]).reshape(NW * NCHUNK, C)
  dst_p = jnp.concatenate([dst, pad_dst]).reshape(NW * NCHUNK, C)
  outf = _sc_scatter(x, src_p, dst_p)
  dp = outf[FBASE:].reshape(NW, N_PAD)
  out = _mlp(outf, dp, W1.T, b1.reshape(1, D), W2.T, b2.reshape(1, D))
  return out[:N_NODES]


# speedup vs baseline: 13.7263x; 1.0433x over previous
import jax
import jax.numpy as jnp
import numpy as np
from jax import lax
from jax.experimental import pallas as pl
from jax.experimental.pallas import tpu as pltpu
from jax.experimental.pallas import tpu_sc as plsc

N_NODES = 10000
D = 128
NC, NS = 2, 16
NW = NC * NS
C = 128
NCHUNK = 80
ROUNDS = 5
CH = NCHUNK // ROUNDS
E_PAD = NW * NCHUNK * C
N_PAD = 10240
TRASH = N_NODES
ROWS_PER_SUB = N_PAD // NS
RCHUNKS = ROWS_PER_SUB // C
HROWS = N_PAD // C
FBASE = NC * N_PAD


def _sc_body(xt, sd, outf, src_v, dst_v, buf0, buf1, hist, acc,
             sem0, sem1, ssem0, ssem1):
  c = lax.axis_index("c")
  s = lax.axis_index("s")
  wid = c * NS + s
  base = s * ROWS_PER_SUB
  z16 = jnp.zeros((16,), jnp.float32)

  @pl.loop(0, C)
  def _(r):
    for k in range(D // 16):
      buf0[r, pl.ds(k * 16, 16)] = z16

  @pl.loop(0, HROWS)
  def _(r):
    for k in range(C // 16):
      hist[r, pl.ds(k * 16, 16)] = z16

  for t in range(RCHUNKS):
    pltpu.sync_copy(buf0, acc.at[pl.ds(base + t * C, C)])

  plsc.subcore_barrier()

  for r in range(ROUNDS):
    pltpu.sync_copy(sd.at[pl.ds(wid * NCHUNK + r * CH, CH)], src_v)
    pltpu.sync_copy(
        sd.at[pl.ds(NW * NCHUNK + wid * NCHUNK + r * CH, CH)], dst_v)

    pltpu.async_copy(xt.at[src_v.at[0]], buf0, sem0)
    pltpu.async_copy(xt.at[src_v.at[1]], buf1, sem1)

    @pl.loop(0, CH // 2)
    def _(i):
      j0 = i * 2
      for b, (buf, sem, ssem) in enumerate(
          ((buf0, sem0, ssem0), (buf1, sem1, ssem1))):
        j = j0 + b
        pltpu.make_async_copy(xt.at[src_v.at[j]], buf, sem).wait()
        pltpu.async_copy(buf, acc.at[dst_v.at[j]], ssem, add=True)
        for k in range(C // 16):
          v = dst_v[j, pl.ds(k * 16, 16)]
          cnt, lastm = plsc.scan_count(v)
          plsc.addupdate_scatter(
              hist,
              [lax.shift_right_logical(v, 7), jnp.bitwise_and(v, 127)],
              cnt.astype(jnp.float32), mask=lastm)
        pltpu.make_async_copy(buf, acc.at[dst_v.at[j]], ssem).wait()
        jn = jnp.minimum(j + 2, CH - 1)
        pltpu.async_copy(xt.at[src_v.at[jn]], buf, sem)

    pltpu.make_async_copy(xt.at[src_v.at[CH - 1]], buf0, sem0).wait()
    pltpu.make_async_copy(xt.at[src_v.at[CH - 1]], buf1, sem1).wait()

  pltpu.sync_copy(hist, outf.at[pl.ds(FBASE + wid * HROWS, HROWS)])

  plsc.subcore_barrier()

  pltpu.sync_copy(acc.at[pl.ds(base, ROWS_PER_SUB)],
                  outf.at[pl.ds(c * N_PAD + base, ROWS_PER_SUB)])


_sc_scatter_cache = []


def _sc_scatter(*args):
  if not _sc_scatter_cache:
    mesh = plsc.VectorSubcoreMesh(
        core_axis_name="c", subcore_axis_name="s",
        num_cores=NC, num_subcores=NS)
    _sc_scatter_cache.append(pl.kernel(
        _sc_body,
        out_type=jax.ShapeDtypeStruct((FBASE + NW * HROWS, D), jnp.float32),
        mesh=mesh,
        compiler_params=pltpu.CompilerParams(needs_layout_passes=False),
        scratch_types=[
            pltpu.VMEM((CH, C), jnp.int32),
            pltpu.VMEM((CH, C), jnp.int32),
            pltpu.VMEM((C, D), jnp.float32),
            pltpu.VMEM((C, D), jnp.float32),
            pltpu.VMEM((HROWS, C), jnp.float32),
            pltpu.VMEM_SHARED((N_PAD, D), jnp.float32),
            pltpu.SemaphoreType.DMA,
            pltpu.SemaphoreType.DMA,
            pltpu.SemaphoreType.DMA,
            pltpu.SemaphoreType.DMA,
        ],
    ))
  return _sc_scatter_cache[0](*args)


NCHK = NW * NCHUNK
PADCH = NCHK - 320000 // C


def _prep_body(e_ref, pad_ref, o_ref):
  o_ref[...] = jnp.concatenate([e_ref[0], pad_ref[0]], axis=0)


def _prep(ei, pads):
  rch = ei.shape[1]
  return pl.pallas_call(
      _prep_body,
      grid=(2,),
      in_specs=[
          pl.BlockSpec((1, rch, C), lambda i: (i, 0, 0)),
          pl.BlockSpec((1, PADCH, C), lambda i: (i, 0, 0)),
      ],
      out_specs=pl.BlockSpec((NCHK, C), lambda i: (i, 0)),
      out_shape=jax.ShapeDtypeStruct((2 * NCHK, C), jnp.int32),
  )(ei, pads)


BN = 1024


def _mlp_body(p0_ref, p1_ref, dp_ref, w1t_ref, b1_ref, w2t_ref, b2_ref,
              o_ref):
  agr = p0_ref[...] + p1_ref[...]
  deg = jnp.sum(dp_ref[...], axis=0)[:, None]
  xn = agr / (deg + 1e-8)
  h = jnp.tanh(
      jnp.dot(xn, w1t_ref[...], preferred_element_type=jnp.float32)
      + b1_ref[...])
  o_ref[...] = (
      jnp.dot(h, w2t_ref[...], preferred_element_type=jnp.float32)
      + b2_ref[...])


def _mlp(outf, dp, w1t, b1, w2t, b2):
  grid = N_PAD // BN
  return pl.pallas_call(
      _mlp_body,
      grid=(grid,),
      in_specs=[
          pl.BlockSpec((BN, D), lambda i: (i, 0)),
          pl.BlockSpec((BN, D), lambda i: (N_PAD // BN + i, 0)),
          pl.BlockSpec((NW, BN), lambda i: (0, i)),
          pl.BlockSpec((D, D), lambda i: (0, 0)),
          pl.BlockSpec((1, D), lambda i: (0, 0)),
          pl.BlockSpec((D, D), lambda i: (0, 0)),
          pl.BlockSpec((1, D), lambda i: (0, 0)),
      ],
      out_specs=pl.BlockSpec((BN, D), lambda i: (i, 0)),
      out_shape=jax.ShapeDtypeStruct((N_PAD, D), jnp.float32),
  )(outf, outf, dp, w1t, b1, w2t, b2)


def kernel(x, edge_index, W1, b1, W2, b2):
  e = edge_index.shape[1]
  ei = edge_index.astype(jnp.int32).reshape(2, e // C, C)
  pad_i = np.arange(PADCH * C, dtype=np.int32)
  pads = jnp.asarray(np.stack([
      pad_i % N_NODES,
      TRASH + pad_i % (N_PAD - N_NODES),
  ]).reshape(2, PADCH, C))
  sd = _prep(ei, pads)
  outf = _sc_scatter(x, sd)
  dp = outf[FBASE:].reshape(NW, N_PAD)
  out = _mlp(outf, dp, W1.T, b1.reshape(1, D), W2.T, b2.reshape(1, D))
  return out[:N_NODES]
